# baseline (speedup 1.0000x reference)
import jax
import jax.numpy as jnp
from jax import lax
from jax.experimental import pallas as pl
from jax.experimental.pallas import tpu as pltpu
from jax.experimental.pallas import tpu_sc as plsc

_N = 64
_TOK = 196
_KEEP = 98
_FRAMES = 4
_D = 768
_LSEQ = 1 + _TOK * _FRAMES
_OUT_TOK = 1 + _KEEP * _FRAMES
_NV = 13
_PADTOK = _NV * 16
_NC = 2
_NS = 16
_NW = _NC * _NS
_ROWS_PER = _N // _NW
_IDX_PAD = 400
_CHUNK = 128


def _body(x_hbm, noise_hbm, xout_hbm, mask_hbm, rest_hbm,
          nbuf, idxbuf, oidxbuf, maskbuf, restbuf, rowbuf, sem):
    lane = lax.iota(jnp.int32, 16)
    wid = lax.axis_index("s") * _NC + lax.axis_index("c")

    for c in range(_FRAMES * _TOK // 16):
        restbuf[pl.ds(c * 16, 16)] = ((lane + c * 16) >> 2) + 1

    def do_row(r, carry):
        i = wid * _ROWS_PER + r
        pltpu.sync_copy(noise_hbm.at[i], nbuf)
        vs = [nbuf[pl.ds(g * 16, 16)] for g in range(_NV)]

        ranks = tuple(jnp.zeros((16,), jnp.int32) for _ in range(_NV))
        for sv in range(_NV):
            def t_step(t, rk, sv=sv):
                k = sv * 16 + t
                b = plsc.load_gather(nbuf, [jnp.full((16,), k, jnp.int32)])
                out = []
                for g in range(_NV):
                    a = vs[g]
                    if g > sv:
                        cond = b <= a
                    elif g < sv:
                        cond = b < a
                    else:
                        cond = jnp.where(lane + g * 16 > k, b <= a, b < a)
                    out.append(rk[g] + cond.astype(jnp.int32))
                return tuple(out)
            ranks = lax.fori_loop(0, 16, t_step, ranks)

        base_idx = i * _LSEQ
        total = jnp.int32(0)
        for g in range(_NV):
            jvec = lane + g * 16
            kept = ranks[g] < _KEEP
            ki = kept.astype(jnp.int32)
            pos = total + plsc.cumsum(ki) - ki
            for f in range(_FRAMES):
                plsc.store_scatter(
                    idxbuf, [pos + (1 + f * _KEEP)],
                    jvec + (base_idx + 1 + f * _TOK), mask=kept)
            total = total + jnp.sum(ki)
            mv = jnp.where(kept, 0.0, 1.0).astype(jnp.float32)
            valid = jvec < _TOK
            for f in range(_FRAMES):
                plsc.store_scatter(maskbuf, [jvec + f * _TOK], mv, mask=valid)
        plsc.store_scatter(idxbuf, [lane],
                           jnp.full((16,), base_idx, jnp.int32),
                           mask=lane == 0)
        plsc.store_scatter(idxbuf, [lane + (_IDX_PAD - 16)],
                           jnp.full((16,), base_idx, jnp.int32),
                           mask=(lane + (_IDX_PAD - 16)) >= _OUT_TOK)

        pltpu.sync_copy(maskbuf, mask_hbm.at[i])
        pltpu.sync_copy(restbuf, rest_hbm.at[i])

        out_base = i * _OUT_TOK
        for c in range(3):
            for t in range(_CHUNK // 16):
                oidxbuf[c, pl.ds(t * 16, 16)] = (
                    out_base + c * _CHUNK + t * 16 + lane)
        tvec = jnp.where(lane < _OUT_TOK - 3 * _CHUNK,
                         out_base + 3 * _CHUNK + lane, out_base)
        pltpu.async_copy(
            x_hbm.at[idxbuf.at[pl.ds(3 * _CHUNK, 16)]],
            rowbuf.at[pl.ds(0, 16)], sem).wait()
        pltpu.async_copy(rowbuf.at[pl.ds(0, 16)], xout_hbm.at[tvec],
                         sem).wait()
        for c in range(3):
            pltpu.async_copy(
                x_hbm.at[idxbuf.at[pl.ds(c * _CHUNK, _CHUNK)]],
                rowbuf, sem).wait()
            pltpu.async_copy(rowbuf, xout_hbm.at[oidxbuf.at[c]],
                             sem).wait()
        return carry

    lax.fori_loop(0, _ROWS_PER, do_row, 0)


@jax.jit
def kernel(x, noise):
    n, l, d = x.shape
    x2 = x.reshape(n * l, d)
    noise_p = jnp.full((_N, _PADTOK), jnp.inf, jnp.float32).at[:, :_TOK].set(noise)
    mesh = plsc.VectorSubcoreMesh(core_axis_name="c", subcore_axis_name="s",
                                  num_cores=_NC, num_subcores=_NS)
    xo, mask, rest = pl.kernel(
        _body,
        out_type=(
            jax.ShapeDtypeStruct((_N * _OUT_TOK, _D), jnp.float32),
            jax.ShapeDtypeStruct((_N, _FRAMES * _TOK), jnp.float32),
            jax.ShapeDtypeStruct((_N, _FRAMES * _TOK), jnp.int32),
        ),
        mesh=mesh,
        compiler_params=pltpu.CompilerParams(needs_layout_passes=False),
        scratch_types=[
            pltpu.VMEM((_PADTOK,), jnp.float32),
            pltpu.VMEM((_IDX_PAD,), jnp.int32),
            pltpu.VMEM((3, _CHUNK), jnp.int32),
            pltpu.VMEM((_FRAMES * _TOK,), jnp.float32),
            pltpu.VMEM((_FRAMES * _TOK,), jnp.int32),
            pltpu.VMEM((_CHUNK, _D), jnp.float32),
            pltpu.SemaphoreType.DMA,
        ],
    )(x2, noise_p)
    return xo.reshape(_N, _OUT_TOK, _D), mask, rest


# speedup vs baseline: 5.3526x; 5.3526x over previous
import jax
import jax.numpy as jnp
from jax import lax
from jax.experimental import pallas as pl
from jax.experimental.pallas import tpu as pltpu
from jax.experimental.pallas import tpu_sc as plsc

_N = 64
_TOK = 196
_KEEP = 98
_FRAMES = 4
_D = 768
_LSEQ = 1 + _TOK * _FRAMES
_OUT_TOK = 1 + _KEEP * _FRAMES
_NV = 13
_PADTOK = _NV * 16
_NC = 2
_NS = 16
_NW = _NC * _NS
_ROWS_PER = _N // _NW
_IDX_PAD = 400
_CHUNK = 80
_NCHUNK = _IDX_PAD // _CHUNK


def _body(x_hbm, noise_hbm, xout_hbm, mask_hbm, rest_hbm,
          nbuf, idx0, idx1, oidx0, oidx1, maskbuf, restbuf, buf,
          gsem, ssem):
    lane = lax.iota(jnp.int32, 16)
    wid = lax.axis_index("s") * _NC + lax.axis_index("c")

    for c in range(_FRAMES * _TOK // 16):
        restbuf[pl.ds(c * 16, 16)] = ((lane + c * 16) >> 2) + 1

    idxb = (idx0, idx1)
    oidxb = (oidx0, oidx1)

    def compute_row(r):
        i = wid * _ROWS_PER + r
        idxbuf = idxb[r]
        oidxbuf = oidxb[r]
        pltpu.sync_copy(noise_hbm.at[i], nbuf)
        vs = [nbuf[pl.ds(g * 16, 16)] for g in range(_NV)]

        ranks = tuple(jnp.zeros((16,), jnp.int32) for _ in range(_NV))
        for sv in range(_NV):
            def t_step(t, rk, sv=sv):
                k = sv * 16 + t
                b = plsc.load_gather(nbuf, [jnp.full((16,), k, jnp.int32)])
                out = []
                for g in range(_NV):
                    a = vs[g]
                    if g > sv:
                        cond = b <= a
                    elif g < sv:
                        cond = b < a
                    else:
                        cond = jnp.where(lane + g * 16 > k, b <= a, b < a)
                    out.append(rk[g] + cond.astype(jnp.int32))
                return tuple(out)
            ranks = lax.fori_loop(0, 16, t_step, ranks)

        total = jnp.int32(0)
        for g in range(_NV):
            jvec = lane + g * 16
            kept = ranks[g] < _KEEP
            ki = kept.astype(jnp.int32)
            pos = total + plsc.cumsum(ki) - ki
            for f in range(_FRAMES):
                plsc.store_scatter(
                    idxbuf, [pos + (1 + f * _KEEP)],
                    jvec * _N + ((1 + f * _TOK) * _N + i), mask=kept)
            total = total + jnp.sum(ki)
            mv = jnp.where(kept, 0.0, 1.0).astype(jnp.float32)
            valid = jvec < _TOK
            for f in range(_FRAMES):
                plsc.store_scatter(maskbuf, [jvec + f * _TOK], mv, mask=valid)
        plsc.store_scatter(idxbuf, [lane],
                           jnp.full((16,), i, jnp.int32),
                           mask=lane == 0)
        plsc.store_scatter(idxbuf, [lane + (_IDX_PAD - 16)],
                           jnp.full((16,), i, jnp.int32),
                           mask=(lane + (_IDX_PAD - 16)) >= _OUT_TOK)

        for c in range(_IDX_PAD // 16):
            q = lane + c * 16
            oidxbuf[pl.ds(c * 16, 16)] = jnp.where(
                q < _OUT_TOK, q * _N + i, i)

        pltpu.sync_copy(maskbuf, mask_hbm.at[i])
        pltpu.sync_copy(restbuf, rest_hbm.at[i])

    prev_scatter = [None, None]

    def issue_gather(r, c, b):
        if prev_scatter[b] is not None:
            prev_scatter[b].wait()
            prev_scatter[b] = None
        return pltpu.async_copy(
            x_hbm.at[idxb[r].at[pl.ds(c * _CHUNK, _CHUNK)]],
            buf.at[b], gsem)

    def drain_row(r, g, base):
        for c in range(_NCHUNK):
            b = (base + c) % 2
            gn = None
            if c + 1 < _NCHUNK:
                gn = issue_gather(r, c + 1, (base + c + 1) % 2)
            g.wait()
            prev_scatter[b] = pltpu.async_copy(
                buf.at[b],
                xout_hbm.at[oidxb[r].at[pl.ds(c * _CHUNK, _CHUNK)]],
                ssem)
            g = gn

    compute_row(0)
    g0 = issue_gather(0, 0, 0)
    drain_row(0, g0, 0)
    compute_row(1)
    g1 = issue_gather(1, 0, 1)
    drain_row(1, g1, 1)
    for s in prev_scatter:
        if s is not None:
            s.wait()


@jax.jit
def kernel(x, noise):
    n, l, d = x.shape
    x2 = x.transpose(1, 0, 2).reshape(l * n, d)
    noise_p = jnp.full((_N, _PADTOK), jnp.inf, jnp.float32).at[:, :_TOK].set(noise)
    mesh = plsc.VectorSubcoreMesh(core_axis_name="c", subcore_axis_name="s",
                                  num_cores=_NC, num_subcores=_NS)
    xo, mask, rest = pl.kernel(
        _body,
        out_type=(
            jax.ShapeDtypeStruct((_OUT_TOK * _N, _D), jnp.float32),
            jax.ShapeDtypeStruct((_N, _FRAMES * _TOK), jnp.float32),
            jax.ShapeDtypeStruct((_N, _FRAMES * _TOK), jnp.int32),
        ),
        mesh=mesh,
        compiler_params=pltpu.CompilerParams(needs_layout_passes=False),
        scratch_types=[
            pltpu.VMEM((_PADTOK,), jnp.float32),
            pltpu.VMEM((_IDX_PAD,), jnp.int32),
            pltpu.VMEM((_IDX_PAD,), jnp.int32),
            pltpu.VMEM((_IDX_PAD,), jnp.int32),
            pltpu.VMEM((_IDX_PAD,), jnp.int32),
            pltpu.VMEM((_FRAMES * _TOK,), jnp.float32),
            pltpu.VMEM((_FRAMES * _TOK,), jnp.int32),
            pltpu.VMEM((2, _CHUNK, _D), jnp.float32),
            pltpu.SemaphoreType.DMA,
            pltpu.SemaphoreType.DMA,
        ],
    )(x2, noise_p)
    return xo.reshape(_OUT_TOK, _N, _D).transpose(1, 0, 2), mask, rest
